# baseline (speedup 1.0000x reference)
import jax
import jax.numpy as jnp
from jax.experimental import pallas as pl
from jax.experimental.pallas import tpu as pltpu

_NCHUNK = 20
_CHUNK = 1600000 // _NCHUNK


def _dma_kernel(u_hbm, b_hbm, ou_hbm, ob_hbm, uv, bv, su, so_u, sin, sout):
    cu_in = pltpu.make_async_copy(u_hbm, uv, su)
    cu_in.start()
    for i in range(_NCHUNK):
        pltpu.make_async_copy(
            b_hbm.at[:, pl.ds(i * _CHUNK, _CHUNK)], bv.at[i], sin.at[i]
        ).start()
    cu_in.wait()
    cu_out = pltpu.make_async_copy(uv, ou_hbm, so_u)
    cu_out.start()
    outs = []
    for i in range(_NCHUNK):
        pltpu.make_async_copy(
            b_hbm.at[:, pl.ds(i * _CHUNK, _CHUNK)], bv.at[i], sin.at[i]
        ).wait()
        c = pltpu.make_async_copy(
            bv.at[i], ob_hbm.at[:, pl.ds(i * _CHUNK, _CHUNK)], sout.at[i]
        )
        c.start()
        outs.append(c)
    cu_out.wait()
    for c in outs:
        c.wait()


def kernel(unary, binary, index1, index2):
    uT = unary.T
    bT = binary.T
    ouT, obT = pl.pallas_call(
        _dma_kernel,
        in_specs=[
            pl.BlockSpec(memory_space=pl.ANY),
            pl.BlockSpec(memory_space=pl.ANY),
        ],
        out_specs=[
            pl.BlockSpec(memory_space=pl.ANY),
            pl.BlockSpec(memory_space=pl.ANY),
        ],
        out_shape=[
            jax.ShapeDtypeStruct(uT.shape, uT.dtype),
            jax.ShapeDtypeStruct(bT.shape, bT.dtype),
        ],
        scratch_shapes=[
            pltpu.VMEM((8, 50000), jnp.float32),
            pltpu.VMEM((_NCHUNK, 2, _CHUNK), jnp.float32),
            pltpu.SemaphoreType.DMA,
            pltpu.SemaphoreType.DMA,
            pltpu.SemaphoreType.DMA((_NCHUNK,)),
            pltpu.SemaphoreType.DMA((_NCHUNK,)),
        ],
    )(uT, bT)
    return ouT.T, obT.T


# speedup vs baseline: 1.0837x; 1.0837x over previous
import jax
import jax.numpy as jnp
from jax.experimental import pallas as pl
from jax.experimental.pallas import tpu as pltpu

_NCHUNK = 5
_CHUNK = 1600000 // _NCHUNK


def _dma_kernel(u_hbm, b_hbm, ou_hbm, ob_hbm, uv, bv, su, so_u, sin, sout):
    cu_in = pltpu.make_async_copy(u_hbm, uv, su)
    cu_in.start()
    for i in range(_NCHUNK):
        pltpu.make_async_copy(
            b_hbm.at[:, pl.ds(i * _CHUNK, _CHUNK)], bv.at[i], sin.at[i]
        ).start()
    cu_in.wait()
    cu_out = pltpu.make_async_copy(uv, ou_hbm, so_u)
    cu_out.start()
    outs = []
    for i in range(_NCHUNK):
        pltpu.make_async_copy(
            b_hbm.at[:, pl.ds(i * _CHUNK, _CHUNK)], bv.at[i], sin.at[i]
        ).wait()
        c = pltpu.make_async_copy(
            bv.at[i], ob_hbm.at[:, pl.ds(i * _CHUNK, _CHUNK)], sout.at[i]
        )
        c.start()
        outs.append(c)
    cu_out.wait()
    for c in outs:
        c.wait()


def kernel(unary, binary, index1, index2):
    uT = unary.T
    bT = binary.T
    ouT, obT = pl.pallas_call(
        _dma_kernel,
        in_specs=[
            pl.BlockSpec(memory_space=pl.ANY),
            pl.BlockSpec(memory_space=pl.ANY),
        ],
        out_specs=[
            pl.BlockSpec(memory_space=pl.ANY),
            pl.BlockSpec(memory_space=pl.ANY),
        ],
        out_shape=[
            jax.ShapeDtypeStruct(uT.shape, uT.dtype),
            jax.ShapeDtypeStruct(bT.shape, bT.dtype),
        ],
        scratch_shapes=[
            pltpu.VMEM((8, 50000), jnp.float32),
            pltpu.VMEM((_NCHUNK, 2, _CHUNK), jnp.float32),
            pltpu.SemaphoreType.DMA,
            pltpu.SemaphoreType.DMA,
            pltpu.SemaphoreType.DMA((_NCHUNK,)),
            pltpu.SemaphoreType.DMA((_NCHUNK,)),
        ],
    )(uT, bT)
    return ouT.T, obT.T


# speedup vs baseline: 1.1016x; 1.0166x over previous
import jax
import jax.numpy as jnp
from jax.experimental import pallas as pl
from jax.experimental.pallas import tpu as pltpu

_NCHUNK = 2
_CHUNK = 1600000 // _NCHUNK


def _dma_kernel(u_hbm, b_hbm, ou_hbm, ob_hbm, uv, bv, su, so_u, sin, sout):
    cu_in = pltpu.make_async_copy(u_hbm, uv, su)
    cu_in.start()
    for i in range(_NCHUNK):
        pltpu.make_async_copy(
            b_hbm.at[:, pl.ds(i * _CHUNK, _CHUNK)], bv.at[i], sin.at[i]
        ).start()
    cu_in.wait()
    cu_out = pltpu.make_async_copy(uv, ou_hbm, so_u)
    cu_out.start()
    outs = []
    for i in range(_NCHUNK):
        pltpu.make_async_copy(
            b_hbm.at[:, pl.ds(i * _CHUNK, _CHUNK)], bv.at[i], sin.at[i]
        ).wait()
        c = pltpu.make_async_copy(
            bv.at[i], ob_hbm.at[:, pl.ds(i * _CHUNK, _CHUNK)], sout.at[i]
        )
        c.start()
        outs.append(c)
    cu_out.wait()
    for c in outs:
        c.wait()


def kernel(unary, binary, index1, index2):
    uT = unary.T
    bT = binary.T
    ouT, obT = pl.pallas_call(
        _dma_kernel,
        in_specs=[
            pl.BlockSpec(memory_space=pl.ANY),
            pl.BlockSpec(memory_space=pl.ANY),
        ],
        out_specs=[
            pl.BlockSpec(memory_space=pl.ANY),
            pl.BlockSpec(memory_space=pl.ANY),
        ],
        out_shape=[
            jax.ShapeDtypeStruct(uT.shape, uT.dtype),
            jax.ShapeDtypeStruct(bT.shape, bT.dtype),
        ],
        scratch_shapes=[
            pltpu.VMEM((8, 50000), jnp.float32),
            pltpu.VMEM((_NCHUNK, 2, _CHUNK), jnp.float32),
            pltpu.SemaphoreType.DMA,
            pltpu.SemaphoreType.DMA,
            pltpu.SemaphoreType.DMA((_NCHUNK,)),
            pltpu.SemaphoreType.DMA((_NCHUNK,)),
        ],
    )(uT, bT)
    return ouT.T, obT.T
